# baseline (speedup 1.0000x reference)
import functools

import jax
import jax.numpy as jnp
from jax import lax
from jax.experimental import pallas as pl
from jax.experimental.pallas import tpu as pltpu
from jax.experimental.pallas import tpu_sc as plsc

_NC = 2
_NS = 16
_NW = _NC * _NS
_CHUNK = 512

_HI = lax.Precision.HIGHEST


def _node_prep_body(x_ref, Wn_ref, bn_ref, We1_ref, Wn1_ref, bn1_ref,
                    glb_ref, Wg_ref, bg_ref, AT_ref, BT_ref, P_ref, H):
    nodes = jnp.dot(x_ref[...], Wn_ref[...],
                    preferred_element_type=jnp.float32,
                    precision=_HI) + bn_ref[...]
    We1 = We1_ref[...]
    Wn1 = Wn1_ref[...]
    g = jnp.dot(glb_ref[...], Wg_ref[...],
                preferred_element_type=jnp.float32, precision=_HI) + bg_ref[...]
    cdim = (((0,), (1,)), ((), ()))
    AT_ref[...] = lax.dot_general(We1[H:2 * H], nodes, cdim,
                                  preferred_element_type=jnp.float32,
                                  precision=_HI)
    BT_ref[...] = lax.dot_general(We1[2 * H:3 * H], nodes, cdim,
                                  preferred_element_type=jnp.float32,
                                  precision=_HI)
    P_ref[...] = (jnp.dot(nodes, Wn1[0:H], preferred_element_type=jnp.float32,
                          precision=_HI)
                  + jnp.dot(g, Wn1[2 * H:3 * H],
                            preferred_element_type=jnp.float32, precision=_HI)
                  + bn1_ref[...])


def _edge_prep_body(eaT_ref, We_ref, be_ref, We1_ref, be1_ref,
                    glb_ref, Wg_ref, bg_ref, baseT_ref, H):
    We1 = We1_ref[...]
    Wc = jnp.dot(We_ref[...], We1[0:H], preferred_element_type=jnp.float32,
                 precision=_HI)
    g = jnp.dot(glb_ref[...], Wg_ref[...],
                preferred_element_type=jnp.float32, precision=_HI) + bg_ref[...]
    cvec = (jnp.dot(be_ref[...], We1[0:H], preferred_element_type=jnp.float32,
                    precision=_HI)
            + jnp.dot(g, We1[3 * H:4 * H], preferred_element_type=jnp.float32,
                      precision=_HI)
            + be1_ref[...])
    cvecT = lax.dot_general(jnp.eye(H, dtype=jnp.float32), cvec,
                            (((1,), (1,)), ((), ())),
                            preferred_element_type=jnp.float32, precision=_HI)
    baseT_ref[...] = lax.dot_general(
        Wc, eaT_ref[...], (((0,), (0,)), ((), ())),
        preferred_element_type=jnp.float32, precision=_HI) + cvecT


def _epilogue_body(agg_ref, P_ref, Wn1_ref, Wn2_ref, bn2_ref, out_ref, H):
    agg2 = agg_ref[...]
    agg = (agg2[0] + agg2[1])[:, :H]
    Wn1 = Wn1_ref[...]
    hn = jnp.maximum(
        P_ref[...] + jnp.dot(agg, Wn1[H:2 * H],
                             preferred_element_type=jnp.float32,
                             precision=_HI), 0.0)
    out_ref[...] = jnp.maximum(
        jnp.dot(hn, Wn2_ref[...], preferred_element_type=jnp.float32,
                precision=_HI) + bn2_ref[...], 0.0)


def _sc_edge_body(A_hbm, B_hbm, base_hbm, s_hbm, r2_hbm, w_hbm, z_hbm,
                  agg_out, A_v, B_v, s_v, r_v, base_v, out_v, w_v,
                  sem_in, sem_out, N_pad, E_pad, per_w, n_chunks, H):
    cid = lax.axis_index("c")
    sid = lax.axis_index("s")
    wid = sid * _NC + cid
    rows_per_tile = N_pad // _NS
    row0 = pl.multiple_of(sid * rows_per_tile, rows_per_tile)
    accum = agg_out.accum

    pltpu.sync_copy(z_hbm.at[pl.ds(row0, rows_per_tile)],
                    accum.at[pl.ds(row0, rows_per_tile)])
    pltpu.sync_copy(A_hbm, A_v)
    pltpu.sync_copy(B_hbm, B_v)
    pltpu.sync_copy(w_hbm, w_v)
    plsc.subcore_barrier()

    iota = lax.iota(jnp.int32, 16)
    wsp = [plsc.load_gather(w_v, [jnp.full((16,), i + 1, jnp.int32)])
           for i in range(H * H + H)]

    groups = _CHUNK // 16
    gper_row = 128 // 16
    nstreams = _CHUNK // 128

    def in_handles(c, b):
        off = wid * per_w + c * _CHUNK
        hs = [pltpu.make_async_copy(
            s_hbm.at[pl.ds(pl.multiple_of(off, _CHUNK), _CHUNK)],
            s_v.at[b], sem_in[b])]
        hs.append(pltpu.make_async_copy(
            r2_hbm.at[pl.ds(pl.multiple_of(off // 128, _CHUNK // 128),
                            _CHUNK // 128)], r_v.at[b], sem_in[b]))
        hs.append(pltpu.make_async_copy(
            base_hbm.at[:, pl.ds(pl.multiple_of(off, _CHUNK), _CHUNK)],
            base_v.at[b], sem_in[b]))
        return hs

    def drain_scatters(b):
        for j in range(nstreams):
            pltpu.make_async_copy(out_v.at[b, pl.ds(j * 128, 128)],
                                  accum.at[r_v.at[b, j]], sem_out[b]).wait()

    for b in range(2):
        for hh in in_handles(b, b):
            hh.start()

    def chunk_body(k, _):
        for b in range(2):
            c = 2 * k + b

            @pl.when(k > 0)
            def _(b=b):
                drain_scatters(b)

            for hh in in_handles(c, b):
                hh.wait()
            for gi in range(groups):
                s16 = s_v[b, pl.ds(gi * 16, 16)]
                r16 = r_v[b, gi // gper_row, pl.ds((gi % gper_row) * 16, 16)]
                h = []
                for ch in range(H):
                    a_c = plsc.load_gather(A_v, [s16 + ch * N_pad])
                    b_c = plsc.load_gather(B_v, [r16 + ch * N_pad])
                    e_c = base_v[b, ch, pl.ds(gi * 16, 16)]
                    h.append(jnp.maximum(a_c + b_c + e_c, 0.0))
                for j in range(H):
                    acc = wsp[H * H + j]
                    for k2 in range(H):
                        acc = acc + h[k2] * wsp[k2 * H + j]
                    acc = jnp.maximum(acc, 0.0)
                    plsc.store_scatter(
                        out_v.at[b],
                        [iota + gi * 16, jnp.full((16,), j, jnp.int32)], acc)
            for j in range(nstreams):
                pltpu.async_copy(out_v.at[b, pl.ds(j * 128, 128)],
                                 accum.at[r_v.at[b, j]], sem_out[b], add=True)

            @pl.when(k < n_chunks // 2 - 1)
            def _(b=b, c=c):
                for hh in in_handles(c + 2, b):
                    hh.start()
        return ()

    lax.fori_loop(0, n_chunks // 2, chunk_body, ())
    for b in range(2):
        drain_scatters(b)
    plsc.subcore_barrier()
    pltpu.sync_copy(accum.at[pl.ds(row0, rows_per_tile)],
                    agg_out.hbm.at[cid, pl.ds(row0, rows_per_tile)])


class _AggRefs:
    def __init__(self, accum, hbm):
        self.accum = accum
        self.hbm = hbm


def _sc_edge_call(A_flat, B_flat, base_flat, s_pad, r2, wvec, zeros_rows,
                  N_pad, E_pad, H):
    per_w = E_pad // _NW
    n_chunks = per_w // _CHUNK
    mesh = plsc.VectorSubcoreMesh(core_axis_name="c", subcore_axis_name="s",
                                  num_cores=_NC, num_subcores=_NS)

    def body(A_hbm, B_hbm, base_hbm, s_hbm, r2_hbm, w_hbm, z_hbm,
             agg_hbm, A_v, B_v, s_v, r_v, base_v, out_v, w_v, accum,
             sem_in0, sem_in1, sem_out0, sem_out1):
        _sc_edge_body(A_hbm, B_hbm, base_hbm, s_hbm, r2_hbm, w_hbm, z_hbm,
                      _AggRefs(accum, agg_hbm), A_v, B_v, s_v, r_v,
                      base_v, out_v, w_v, (sem_in0, sem_in1),
                      (sem_out0, sem_out1), N_pad, E_pad, per_w, n_chunks, H)

    f = pl.kernel(
        body,
        out_type=jax.ShapeDtypeStruct((_NC, N_pad, 8), jnp.float32),
        mesh=mesh,
        compiler_params=pltpu.CompilerParams(needs_layout_passes=False,
                                             use_tc_tiling_on_sc=False),
        scratch_types=[
            pltpu.VMEM((N_pad * H,), jnp.float32),
            pltpu.VMEM((N_pad * H,), jnp.float32),
            pltpu.VMEM((2, _CHUNK), jnp.int32),
            pltpu.VMEM((2, _CHUNK // 128, 128), jnp.int32),
            pltpu.VMEM((2, H, _CHUNK), jnp.float32),
            pltpu.VMEM((2, _CHUNK, 8), jnp.float32),
            pltpu.VMEM((32,), jnp.float32),
            pltpu.VMEM_SHARED((N_pad, 8), jnp.float32),
            pltpu.SemaphoreType.DMA,
            pltpu.SemaphoreType.DMA,
            pltpu.SemaphoreType.DMA,
            pltpu.SemaphoreType.DMA,
        ],
    )
    return f(A_flat, B_flat, base_flat, s_pad, r2, wvec, zeros_rows)


def kernel(x, edge_attr, globals_, edge_index, Wn, bn, We, be, Wg, bg,
           We1, be1, We2, be2, Wn1, bn1, Wn2, bn2):
    N, DF = x.shape
    E, DE = edge_attr.shape
    H = Wn.shape[1]

    BN = 2048
    BE = 32768
    N_pad = -((N + 1) // -1024) * 1024
    N_pad = -(N_pad // -BN) * BN if N_pad % BN else N_pad
    E_pad = -(E // -(_NW * _CHUNK)) * (_NW * _CHUNK)

    s_pad = jnp.concatenate(
        [edge_index[0], jnp.zeros((E_pad - E,), jnp.int32)])
    r_pad = jnp.concatenate(
        [edge_index[1], jnp.full((E_pad - E,), N, jnp.int32)])
    r2 = r_pad.reshape(E_pad // 128, 128)

    eaT = edge_attr.T

    bn_2d = bn.reshape(1, H)
    be_2d = be.reshape(1, H)
    bg_2d = bg.reshape(1, H)
    be1_2d = be1.reshape(1, H)
    bn1_2d = bn1.reshape(1, H)
    bn2_2d = bn2.reshape(1, H)

    full = lambda a: pl.BlockSpec(a.shape, lambda i: (0,) * a.ndim)

    AT, BT, P = pl.pallas_call(
        functools.partial(_node_prep_body, H=H),
        grid=(N_pad // BN,),
        in_specs=[
            pl.BlockSpec((BN, DF), lambda i: (i, 0)),
            full(Wn), full(bn_2d), full(We1), full(Wn1), full(bn1_2d),
            full(globals_), full(Wg), full(bg_2d),
        ],
        out_specs=[
            pl.BlockSpec((H, BN), lambda i: (0, i)),
            pl.BlockSpec((H, BN), lambda i: (0, i)),
            pl.BlockSpec((BN, H), lambda i: (i, 0)),
        ],
        out_shape=[
            jax.ShapeDtypeStruct((H, N_pad), jnp.float32),
            jax.ShapeDtypeStruct((H, N_pad), jnp.float32),
            jax.ShapeDtypeStruct((N_pad, H), jnp.float32),
        ],
    )(x, Wn, bn_2d, We1, Wn1, bn1_2d, globals_, Wg, bg_2d)

    baseT = pl.pallas_call(
        functools.partial(_edge_prep_body, H=H),
        grid=(E_pad // BE,),
        in_specs=[
            pl.BlockSpec((DE, BE), lambda i: (0, i)),
            full(We), full(be_2d), full(We1), full(be1_2d),
            full(globals_), full(Wg), full(bg_2d),
        ],
        out_specs=pl.BlockSpec((H, BE), lambda i: (0, i)),
        out_shape=jax.ShapeDtypeStruct((H, E_pad), jnp.float32),
    )(eaT, We, be_2d, We1, be1_2d, globals_, Wg, bg_2d)

    wvec_n = (1 + H * H + H + 15) // 16 * 16
    wvec = jnp.concatenate(
        [jnp.zeros((1,), jnp.float32), We2.reshape(-1), be2,
         jnp.zeros((wvec_n - 1 - H * H - H,), jnp.float32)])
    zeros_rows = jnp.zeros((N_pad, 8), jnp.float32)
    agg2 = _sc_edge_call(AT.reshape(-1), BT.reshape(-1), baseT,
                         s_pad, r2, wvec, zeros_rows, N_pad, E_pad, H)

    out = pl.pallas_call(
        functools.partial(_epilogue_body, H=H),
        grid=(N_pad // BN,),
        in_specs=[
            pl.BlockSpec((_NC, BN, 8), lambda i: (0, i, 0)),
            pl.BlockSpec((BN, H), lambda i: (i, 0)),
            full(Wn1), full(Wn2), full(bn2_2d),
        ],
        out_specs=pl.BlockSpec((BN, H), lambda i: (i, 0)),
        out_shape=jax.ShapeDtypeStruct((N_pad, H), jnp.float32),
    )(agg2, P, Wn1, Wn2, bn2_2d)

    return out[:N]


# speedup vs baseline: 1.0057x; 1.0057x over previous
import functools

import jax
import jax.numpy as jnp
from jax import lax
from jax.experimental import pallas as pl
from jax.experimental.pallas import tpu as pltpu
from jax.experimental.pallas import tpu_sc as plsc

_NC = 2
_NS = 16
_NW = _NC * _NS
_CHUNK = 512

_HI = lax.Precision.HIGHEST


def _node_prep_body(x_ref, Wn_ref, bn_ref, We1_ref, Wn1_ref, bn1_ref,
                    glb_ref, Wg_ref, bg_ref, AT_ref, BT_ref, P_ref, H):
    nodes = jnp.dot(x_ref[...], Wn_ref[...],
                    preferred_element_type=jnp.float32,
                    precision=_HI) + bn_ref[...]
    We1 = We1_ref[...]
    Wn1 = Wn1_ref[...]
    g = jnp.dot(glb_ref[...], Wg_ref[...],
                preferred_element_type=jnp.float32, precision=_HI) + bg_ref[...]
    cdim = (((0,), (1,)), ((), ()))
    AT_ref[...] = lax.dot_general(We1[H:2 * H], nodes, cdim,
                                  preferred_element_type=jnp.float32,
                                  precision=_HI)
    BT_ref[...] = lax.dot_general(We1[2 * H:3 * H], nodes, cdim,
                                  preferred_element_type=jnp.float32,
                                  precision=_HI)
    P_ref[...] = (jnp.dot(nodes, Wn1[0:H], preferred_element_type=jnp.float32,
                          precision=_HI)
                  + jnp.dot(g, Wn1[2 * H:3 * H],
                            preferred_element_type=jnp.float32, precision=_HI)
                  + bn1_ref[...])


def _edge_prep_body(eaT_ref, We_ref, be_ref, We1_ref, be1_ref,
                    glb_ref, Wg_ref, bg_ref, baseT_ref, H):
    We1 = We1_ref[...]
    Wc = jnp.dot(We_ref[...], We1[0:H], preferred_element_type=jnp.float32,
                 precision=_HI)
    g = jnp.dot(glb_ref[...], Wg_ref[...],
                preferred_element_type=jnp.float32, precision=_HI) + bg_ref[...]
    cvec = (jnp.dot(be_ref[...], We1[0:H], preferred_element_type=jnp.float32,
                    precision=_HI)
            + jnp.dot(g, We1[3 * H:4 * H], preferred_element_type=jnp.float32,
                      precision=_HI)
            + be1_ref[...])
    cvecT = lax.dot_general(jnp.eye(H, dtype=jnp.float32), cvec,
                            (((1,), (1,)), ((), ())),
                            preferred_element_type=jnp.float32, precision=_HI)
    baseT_ref[...] = lax.dot_general(
        Wc, eaT_ref[...], (((0,), (0,)), ((), ())),
        preferred_element_type=jnp.float32, precision=_HI) + cvecT


def _epilogue_body(agg_ref, P_ref, Wn1_ref, Wn2_ref, bn2_ref, out_ref, H):
    agg2 = agg_ref[...]
    agg = (agg2[0] + agg2[1])[:, :H]
    Wn1 = Wn1_ref[...]
    hn = jnp.maximum(
        P_ref[...] + jnp.dot(agg, Wn1[H:2 * H],
                             preferred_element_type=jnp.float32,
                             precision=_HI), 0.0)
    out_ref[...] = jnp.maximum(
        jnp.dot(hn, Wn2_ref[...], preferred_element_type=jnp.float32,
                precision=_HI) + bn2_ref[...], 0.0)


def _sc_edge_body(A_hbm, B_hbm, base_hbm, s_hbm, r2_hbm, w_hbm, z_hbm,
                  agg_out, A_v, B_v, s_v, r_v, base_v, out_v, w_v,
                  sem_in, sem_out, N_pad, E_pad, per_w, n_chunks, H):
    cid = lax.axis_index("c")
    sid = lax.axis_index("s")
    wid = sid * _NC + cid
    rows_per_tile = N_pad // _NS
    row0 = pl.multiple_of(sid * rows_per_tile, rows_per_tile)
    accum = agg_out.accum

    pltpu.sync_copy(z_hbm.at[pl.ds(row0, rows_per_tile)],
                    accum.at[pl.ds(row0, rows_per_tile)])
    pltpu.sync_copy(A_hbm, A_v)
    pltpu.sync_copy(B_hbm, B_v)
    pltpu.sync_copy(w_hbm, w_v)
    plsc.subcore_barrier()

    iota = lax.iota(jnp.int32, 16)
    wsp = [plsc.load_gather(w_v, [jnp.full((16,), i + 1, jnp.int32)])
           for i in range(H * H + H)]

    groups = _CHUNK // 16
    gper_row = 128 // 16
    nstreams = _CHUNK // 128

    def in_handles(c, b):
        off = wid * per_w + c * _CHUNK
        hs = [pltpu.make_async_copy(
            s_hbm.at[pl.ds(pl.multiple_of(off, _CHUNK), _CHUNK)],
            s_v.at[b], sem_in[b])]
        hs.append(pltpu.make_async_copy(
            r2_hbm.at[pl.ds(pl.multiple_of(off // 128, _CHUNK // 128),
                            _CHUNK // 128)], r_v.at[b], sem_in[b]))
        hs.append(pltpu.make_async_copy(
            base_hbm.at[:, pl.ds(pl.multiple_of(off, _CHUNK), _CHUNK)],
            base_v.at[b], sem_in[b]))
        return hs

    def drain_scatters(b):
        for j in range(nstreams):
            pltpu.make_async_copy(out_v.at[b, pl.ds(j * 128, 128)],
                                  accum.at[r_v.at[b, j]], sem_out[b]).wait()

    for b in range(2):
        for hh in in_handles(b, b):
            hh.start()

    def chunk_body(k, _):
        for b in range(2):
            c = 2 * k + b

            @pl.when(k > 0)
            def _(b=b):
                drain_scatters(b)

            for hh in in_handles(c, b):
                hh.wait()
            for gi in range(groups):
                s16 = s_v[b, pl.ds(gi * 16, 16)]
                r16 = r_v[b, gi // gper_row, pl.ds((gi % gper_row) * 16, 16)]
                h = []
                for ch in range(H):
                    a_c = plsc.load_gather(A_v, [s16 + ch * N_pad])
                    b_c = plsc.load_gather(B_v, [r16 + ch * N_pad])
                    e_c = base_v[b, ch, pl.ds(gi * 16, 16)]
                    h.append(jnp.maximum(a_c + b_c + e_c, 0.0))
                for j in range(H):
                    acc = wsp[H * H + j]
                    for k2 in range(H):
                        acc = acc + h[k2] * wsp[k2 * H + j]
                    acc = jnp.maximum(acc, 0.0)
                    plsc.store_scatter(
                        out_v.at[b],
                        [iota + gi * 16, jnp.full((16,), j, jnp.int32)], acc)
            for j in range(nstreams):
                pltpu.async_copy(out_v.at[b, pl.ds(j * 128, 128)],
                                 accum.at[r_v.at[b, j]], sem_out[b], add=True)

            @pl.when(k < n_chunks // 2 - 1)
            def _(b=b, c=c):
                for hh in in_handles(c + 2, b):
                    hh.start()
        return ()

    lax.fori_loop(0, n_chunks // 2, chunk_body, ())
    for b in range(2):
        drain_scatters(b)
    plsc.subcore_barrier()
    pltpu.sync_copy(accum.at[pl.ds(row0, rows_per_tile)],
                    agg_out.hbm.at[cid, pl.ds(row0, rows_per_tile)])


class _AggRefs:
    def __init__(self, accum, hbm):
        self.accum = accum
        self.hbm = hbm


def _sc_edge_call(A_flat, B_flat, baseT, s_pad, r2, wvec, zeros_rows,
                  N_pad, E_pad, H):
    per_w = E_pad // _NW
    n_chunks = per_w // _CHUNK
    mesh = plsc.VectorSubcoreMesh(core_axis_name="c", subcore_axis_name="s",
                                  num_cores=_NC, num_subcores=_NS)

    def body(A_hbm, B_hbm, base_hbm, s_hbm, r2_hbm, w_hbm, z_hbm,
             agg_hbm, A_v, B_v, s_v, r_v, base_v, out_v, w_v, accum,
             sem_in0, sem_in1, sem_out0, sem_out1):
        _sc_edge_body(A_hbm, B_hbm, base_hbm, s_hbm, r2_hbm, w_hbm, z_hbm,
                      _AggRefs(accum, agg_hbm), A_v, B_v, s_v, r_v,
                      base_v, out_v, w_v, (sem_in0, sem_in1),
                      (sem_out0, sem_out1), N_pad, E_pad, per_w, n_chunks, H)

    f = pl.kernel(
        body,
        out_type=jax.ShapeDtypeStruct((_NC, N_pad, 8), jnp.float32),
        mesh=mesh,
        compiler_params=pltpu.CompilerParams(needs_layout_passes=False,
                                             use_tc_tiling_on_sc=False),
        scratch_types=[
            pltpu.VMEM((N_pad * H,), jnp.float32),
            pltpu.VMEM((N_pad * H,), jnp.float32),
            pltpu.VMEM((2, _CHUNK), jnp.int32),
            pltpu.VMEM((2, _CHUNK // 128, 128), jnp.int32),
            pltpu.VMEM((2, H, _CHUNK), jnp.float32),
            pltpu.VMEM((2, _CHUNK, 8), jnp.float32),
            pltpu.VMEM((32,), jnp.float32),
            pltpu.VMEM_SHARED((N_pad, 8), jnp.float32),
            pltpu.SemaphoreType.DMA,
            pltpu.SemaphoreType.DMA,
            pltpu.SemaphoreType.DMA,
            pltpu.SemaphoreType.DMA,
        ],
    )
    return f(A_flat, B_flat, baseT, s_pad, r2, wvec, zeros_rows)


def kernel(x, edge_attr, globals_, edge_index, Wn, bn, We, be, Wg, bg,
           We1, be1, We2, be2, Wn1, bn1, Wn2, bn2):
    N, DF = x.shape
    E, DE = edge_attr.shape
    H = Wn.shape[1]

    BN = 2048
    BE = 32768
    N_pad = -((N + 1) // -1024) * 1024
    N_pad = -(N_pad // -BN) * BN if N_pad % BN else N_pad
    E_pad = -(E // -(_NW * _CHUNK)) * (_NW * _CHUNK)

    s_pad = jnp.concatenate(
        [edge_index[0], jnp.zeros((E_pad - E,), jnp.int32)])
    r_pad = jnp.concatenate(
        [edge_index[1], jnp.full((E_pad - E,), N, jnp.int32)])
    r2 = r_pad.reshape(E_pad // 128, 128)

    eaT = edge_attr.T

    bn_2d = bn.reshape(1, H)
    be_2d = be.reshape(1, H)
    bg_2d = bg.reshape(1, H)
    be1_2d = be1.reshape(1, H)
    bn1_2d = bn1.reshape(1, H)
    bn2_2d = bn2.reshape(1, H)

    full = lambda a: pl.BlockSpec(a.shape, lambda i: (0,) * a.ndim)

    AT, BT, P = pl.pallas_call(
        functools.partial(_node_prep_body, H=H),
        grid=(N_pad // BN,),
        in_specs=[
            pl.BlockSpec((BN, DF), lambda i: (i, 0)),
            full(Wn), full(bn_2d), full(We1), full(Wn1), full(bn1_2d),
            full(globals_), full(Wg), full(bg_2d),
        ],
        out_specs=[
            pl.BlockSpec((H, BN), lambda i: (0, i)),
            pl.BlockSpec((H, BN), lambda i: (0, i)),
            pl.BlockSpec((BN, H), lambda i: (i, 0)),
        ],
        out_shape=[
            jax.ShapeDtypeStruct((H, N_pad), jnp.float32),
            jax.ShapeDtypeStruct((H, N_pad), jnp.float32),
            jax.ShapeDtypeStruct((N_pad, H), jnp.float32),
        ],
    )(x, Wn, bn_2d, We1, Wn1, bn1_2d, globals_, Wg, bg_2d)

    baseT = pl.pallas_call(
        functools.partial(_edge_prep_body, H=H),
        grid=(E_pad // BE,),
        in_specs=[
            pl.BlockSpec((DE, BE), lambda i: (0, i)),
            full(We), full(be_2d), full(We1), full(be1_2d),
            full(globals_), full(Wg), full(bg_2d),
        ],
        out_specs=pl.BlockSpec((H, BE), lambda i: (0, i)),
        out_shape=jax.ShapeDtypeStruct((H, E_pad), jnp.float32),
    )(eaT, We, be_2d, We1, be1_2d, globals_, Wg, bg_2d)

    wvec_n = (1 + H * H + H + 15) // 16 * 16
    wvec = jnp.concatenate(
        [jnp.zeros((1,), jnp.float32), We2.reshape(-1), be2,
         jnp.zeros((wvec_n - 1 - H * H - H,), jnp.float32)])
    zeros_rows = jnp.zeros((N_pad, 8), jnp.float32)
    agg2 = _sc_edge_call(AT.reshape(-1), BT.reshape(-1), baseT,
                         s_pad, r2, wvec, zeros_rows, N_pad, E_pad, H)

    out = pl.pallas_call(
        functools.partial(_epilogue_body, H=H),
        grid=(N_pad // BN,),
        in_specs=[
            pl.BlockSpec((_NC, BN, 8), lambda i: (0, i, 0)),
            pl.BlockSpec((BN, H), lambda i: (i, 0)),
            full(Wn1), full(Wn2), full(bn2_2d),
        ],
        out_specs=pl.BlockSpec((BN, H), lambda i: (i, 0)),
        out_shape=jax.ShapeDtypeStruct((N_pad, H), jnp.float32),
    )(agg2, P, Wn1, Wn2, bn2_2d)

    return out[:N]
